# Initial kernel scaffold; baseline (speedup 1.0000x reference)
import jax
import jax.numpy as jnp
from jax.experimental import pallas as pl


def kernel(coords, edge_index):
    raise NotImplementedError("write your pallas kernel here")


# speedup vs baseline: 5.3774x; 5.3774x over previous
import dataclasses
import functools

import jax
import jax.numpy as jnp
import numpy as np
from jax import lax
from jax.experimental import pallas as pl
from jax.experimental.pallas import tpu as pltpu
from jax.experimental.pallas import tpu_sc as plsc

_NUM_BASES = 16
_D_MIN = 0.0
_D_MAX = 4.5
_WIDTH = (_D_MAX - _D_MIN) / _NUM_BASES
_INV_2W2 = 1.0 / (2.0 * _WIDTH * _WIDTH)

_NC = 2
_NS = 16
_NW = _NC * _NS

_W = 800
_SUB = 80

_MAGIC = np.int32(0x5F3759DF)


def _sc_body(coords_hbm, src_hbm, dst_hbm, d_hbm, vec_hbm,
             idx_s, idx_d, rows_s, rows_d, dloc, vloc, sem):
    E = src_hbm.shape[0]
    per_w = E // _NW
    cid = lax.axis_index("c")
    sid = lax.axis_index("s")
    wid = sid * _NC + cid

    iota16 = lax.iota(jnp.int32, 16)
    col0 = jnp.zeros((16,), jnp.int32)
    col1 = jnp.full((16,), 1, jnp.int32)
    col2 = jnp.full((16,), 2, jnp.int32)

    base0 = wid * per_w

    @pl.loop(0, per_w, step=_W)
    def _window(off):
        base = base0 + off
        pltpu.sync_copy(src_hbm.at[pl.ds(base, _W)], idx_s)
        pltpu.sync_copy(dst_hbm.at[pl.ds(base, _W)], idx_d)

        handles = []
        for j in range(_W // _SUB):
            sl = pl.ds(j * _SUB, _SUB)
            handles.append(
                pltpu.async_copy(coords_hbm.at[idx_s.at[sl]], rows_s.at[sl], sem))
            handles.append(
                pltpu.async_copy(coords_hbm.at[idx_d.at[sl]], rows_d.at[sl], sem))
        for h in handles:
            h.wait()

        @pl.loop(0, _W, step=16)
        def _edges(e0):
            ri = iota16 + e0
            xs = plsc.load_gather(rows_s, [ri, col0])
            ys = plsc.load_gather(rows_s, [ri, col1])
            zs = plsc.load_gather(rows_s, [ri, col2])
            xd = plsc.load_gather(rows_d, [ri, col0])
            yd = plsc.load_gather(rows_d, [ri, col1])
            zd = plsc.load_gather(rows_d, [ri, col2])
            dx = xd - xs
            dy = yd - ys
            dz = zd - zs
            ss = jnp.maximum(dx * dx + dy * dy + dz * dz, 1e-24)
            yi = _MAGIC - lax.shift_right_logical(plsc.bitcast(ss, jnp.int32), 1)
            y = plsc.bitcast(yi, jnp.float32)
            h = 0.5 * ss
            y = y * (1.5 - h * y * y)
            y = y * (1.5 - h * y * y)
            y = y * (1.5 - h * y * y)
            dloc[pl.ds(e0, 16)] = ss * y
            plsc.store_scatter(vloc, [ri, col0], dx * y)
            plsc.store_scatter(vloc, [ri, col1], dy * y)
            plsc.store_scatter(vloc, [ri, col2], dz * y)

        pltpu.sync_copy(dloc, d_hbm.at[pl.ds(base, _W)])
        pltpu.sync_copy(vloc, vec_hbm.at[pl.ds(base, _W)])


@functools.partial(jax.jit, static_argnames=())
def _sc_gather_norm(coords4, src, dst):
    E = src.shape[0]
    mesh = plsc.VectorSubcoreMesh(core_axis_name="c", subcore_axis_name="s")
    cp = pltpu.CompilerParams()
    if "needs_layout_passes" in pltpu.CompilerParams.__dataclass_fields__:
        cp = dataclasses.replace(cp, needs_layout_passes=False)
    if "use_tc_tiling_on_sc" in pltpu.CompilerParams.__dataclass_fields__:
        cp = dataclasses.replace(cp, use_tc_tiling_on_sc=False)
    kern = pl.kernel(
        _sc_body,
        out_type=[
            jax.ShapeDtypeStruct((E,), jnp.float32),
            jax.ShapeDtypeStruct((E, 3), jnp.float32),
        ],
        mesh=mesh,
        scratch_types=[
            pltpu.VMEM((_W,), jnp.int32),
            pltpu.VMEM((_W,), jnp.int32),
            pltpu.VMEM((_W, 16), jnp.float32),
            pltpu.VMEM((_W, 16), jnp.float32),
            pltpu.VMEM((_W,), jnp.float32),
            pltpu.VMEM((_W, 3), jnp.float32),
            pltpu.SemaphoreType.DMA,
        ],
        compiler_params=cp,
    )
    return kern(coords4, src, dst)


_CENTERS = np.linspace(_D_MIN, _D_MAX, _NUM_BASES, dtype=np.float32)
_C_STEP = float(_CENTERS[1] - _CENTERS[0])


def _rbf_body(d8_ref, out_ref):
    dp = d8_ref[...]
    lane = lax.broadcasted_iota(jnp.int32, (8, 128), 1)
    krow = lax.broadcasted_iota(jnp.int32, (8, 128), 0)
    m = jnp.where(lax.shift_right_logical(lane, 4) == krow, 1.0, 0.0)
    c128 = _C_STEP * (lane[:1] & 15).astype(jnp.float32)
    D = jnp.dot(dp, m, precision=jax.lax.Precision.HIGHEST,
                preferred_element_type=jnp.float32)
    t = D - c128
    out_ref[...] = jnp.exp(t * t * (-_INV_2W2))


def _rbf(d8):
    R = d8.shape[0]
    bn = 1000
    return pl.pallas_call(
        _rbf_body,
        grid=(R // bn,),
        in_specs=[pl.BlockSpec((bn, 8), lambda i: (i, 0))],
        out_specs=pl.BlockSpec((bn, 128), lambda i: (i, 0)),
        out_shape=jax.ShapeDtypeStruct((R, 128), jnp.float32),
    )(d8)


def kernel(coords, edge_index):
    N = coords.shape[0]
    E = edge_index.shape[1]
    coords4 = jnp.pad(coords, ((0, 0), (0, 13)))
    src = edge_index[0]
    dst = edge_index[1]
    d, vec = _sc_gather_norm(coords4, src, dst)
    scal = _rbf(d.reshape(E // 8, 8))
    return scal.reshape(E, _NUM_BASES), vec.reshape(E, 3, 1)


# speedup vs baseline: 5.9269x; 1.1022x over previous
import dataclasses
import functools

import jax
import jax.numpy as jnp
import numpy as np
from jax import lax
from jax.experimental import pallas as pl
from jax.experimental.pallas import tpu as pltpu
from jax.experimental.pallas import tpu_sc as plsc

_NUM_BASES = 16
_D_MIN = 0.0
_D_MAX = 4.5
_WIDTH = (_D_MAX - _D_MIN) / _NUM_BASES
_INV_2W2 = 1.0 / (2.0 * _WIDTH * _WIDTH)

_NC = 2
_NS = 16
_NW = _NC * _NS

_W = 800
_SUB = 80

_MAGIC = np.int32(0x5F3759DF)


def _sc_body(coords_hbm, src_hbm, dst_hbm, d_hbm, vec_hbm,
             idx_s, idx_d, rows_s, rows_d, dloc, vloc, sem):
    E = src_hbm.shape[0]
    per_w = E // _NW
    cid = lax.axis_index("c")
    sid = lax.axis_index("s")
    wid = sid * _NC + cid

    table = coords_hbm

    iota16 = lax.iota(jnp.int32, 16)
    col0 = jnp.zeros((16,), jnp.int32)
    col1 = jnp.full((16,), 1, jnp.int32)
    col2 = jnp.full((16,), 2, jnp.int32)

    base0 = wid * per_w

    @pl.loop(0, per_w, step=_W)
    def _window(off):
        base = base0 + off
        pltpu.sync_copy(src_hbm.at[pl.ds(base, _W)], idx_s)
        pltpu.sync_copy(dst_hbm.at[pl.ds(base, _W)], idx_d)

        handles = []
        for j in range(_W // _SUB):
            sl = pl.ds(j * _SUB, _SUB)
            handles.append(
                pltpu.async_copy(table.at[idx_s.at[sl]], rows_s.at[sl], sem))
            handles.append(
                pltpu.async_copy(table.at[idx_d.at[sl]], rows_d.at[sl], sem))
        for h in handles:
            h.wait()

        @pl.loop(0, _W, step=16)
        def _edges(e0):
            ri = iota16 + e0
            xs = plsc.load_gather(rows_s, [ri, col0])
            ys = plsc.load_gather(rows_s, [ri, col1])
            zs = plsc.load_gather(rows_s, [ri, col2])
            xd = plsc.load_gather(rows_d, [ri, col0])
            yd = plsc.load_gather(rows_d, [ri, col1])
            zd = plsc.load_gather(rows_d, [ri, col2])
            dx = xd - xs
            dy = yd - ys
            dz = zd - zs
            ss = jnp.maximum(dx * dx + dy * dy + dz * dz, 1e-24)
            yi = _MAGIC - lax.shift_right_logical(plsc.bitcast(ss, jnp.int32), 1)
            y = plsc.bitcast(yi, jnp.float32)
            h = 0.5 * ss
            y = y * (1.5 - h * y * y)
            y = y * (1.5 - h * y * y)
            y = y * (1.5 - h * y * y)
            dloc[pl.ds(e0, 16)] = ss * y
            fi = ri + (ri + ri)
            plsc.store_scatter(vloc, [fi], dx * y)
            plsc.store_scatter(vloc, [fi + col1], dy * y)
            plsc.store_scatter(vloc, [fi + col2], dz * y)

        pltpu.sync_copy(dloc, d_hbm.at[pl.ds(base, _W)])
        pltpu.sync_copy(vloc, vec_hbm.at[pl.ds(3 * base, 3 * _W)])


@functools.partial(jax.jit, static_argnames=())
def _sc_gather_norm(coords4, src, dst):
    E = src.shape[0]
    mesh = plsc.VectorSubcoreMesh(core_axis_name="c", subcore_axis_name="s")
    cp = pltpu.CompilerParams()
    if "needs_layout_passes" in pltpu.CompilerParams.__dataclass_fields__:
        cp = dataclasses.replace(cp, needs_layout_passes=False)
    if "use_tc_tiling_on_sc" in pltpu.CompilerParams.__dataclass_fields__:
        cp = dataclasses.replace(cp, use_tc_tiling_on_sc=False)
    kern = pl.kernel(
        _sc_body,
        out_type=[
            jax.ShapeDtypeStruct((E,), jnp.float32),
            jax.ShapeDtypeStruct((3 * E,), jnp.float32),
        ],
        mesh=mesh,
        scratch_types=[
            pltpu.VMEM((_W,), jnp.int32),
            pltpu.VMEM((_W,), jnp.int32),
            pltpu.VMEM((_W, 16), jnp.float32),
            pltpu.VMEM((_W, 16), jnp.float32),
            pltpu.VMEM((_W,), jnp.float32),
            pltpu.VMEM((3 * _W,), jnp.float32),
            pltpu.SemaphoreType.DMA,
        ],
        compiler_params=cp,
    )
    return kern(coords4, src, dst)


_CENTERS = np.linspace(_D_MIN, _D_MAX, _NUM_BASES, dtype=np.float32)
_C_STEP = float(_CENTERS[1] - _CENTERS[0])


def _rbf_body(d8_ref, out_ref):
    dp = d8_ref[...]
    lane = lax.broadcasted_iota(jnp.int32, (8, 128), 1)
    krow = lax.broadcasted_iota(jnp.int32, (8, 128), 0)
    m = jnp.where(lax.shift_right_logical(lane, 4) == krow, 1.0, 0.0)
    c128 = _C_STEP * (lane[:1] & 15).astype(jnp.float32)
    D = jnp.dot(dp, m, precision=jax.lax.Precision.HIGHEST,
                preferred_element_type=jnp.float32)
    t = D - c128
    out_ref[...] = jnp.exp(t * t * (-_INV_2W2))


def _rbf(d8):
    R = d8.shape[0]
    bn = 1000
    return pl.pallas_call(
        _rbf_body,
        grid=(R // bn,),
        in_specs=[pl.BlockSpec((bn, 8), lambda i: (i, 0))],
        out_specs=pl.BlockSpec((bn, 128), lambda i: (i, 0)),
        out_shape=jax.ShapeDtypeStruct((R, 128), jnp.float32),
    )(d8)


def kernel(coords, edge_index):
    N = coords.shape[0]
    E = edge_index.shape[1]
    coords16 = jnp.pad(coords, ((0, 0), (0, 13)))
    src = edge_index[0]
    dst = edge_index[1]
    d, vec = _sc_gather_norm(coords16, src, dst)
    scal = _rbf(d.reshape(E // 8, 8))
    return scal.reshape(E, _NUM_BASES), vec.reshape(E, 3, 1)


# speedup vs baseline: 20.9697x; 3.5381x over previous
import dataclasses
import functools

import jax
import jax.numpy as jnp
import numpy as np
from jax import lax
from jax.experimental import pallas as pl
from jax.experimental.pallas import tpu as pltpu
from jax.experimental.pallas import tpu_sc as plsc

_NUM_BASES = 16
_D_MIN = 0.0
_D_MAX = 4.5
_WIDTH = (_D_MAX - _D_MIN) / _NUM_BASES
_INV_2W2 = 1.0 / (2.0 * _WIDTH * _WIDTH)
_C_STEP = (_D_MAX - _D_MIN) / (_NUM_BASES - 1)

_NC = 2
_NS = 16
_NW = _NC * _NS

_W = 640
_WROWS = _W // 128

_MAGIC = np.int32(0x5F3759DF)


def _sc_body(coords_hbm, src_hbm, dst_hbm, d_hbm, vec_hbm,
             idx_s, idx_d, rows_s, rows_d, dloc, vloc, sem):
    E = src_hbm.shape[0] * 128
    nwin = E // _W
    wpt = (nwin + _NW - 1) // _NW
    cid = lax.axis_index("c")
    sid = lax.axis_index("s")
    wid = sid * _NC + cid

    iota16 = lax.iota(jnp.int32, 16)
    col0 = jnp.zeros((16,), jnp.int32)
    col1 = jnp.full((16,), 1, jnp.int32)
    col2 = jnp.full((16,), 2, jnp.int32)
    lmask = jnp.full((16,), 127, jnp.int32)

    @pl.loop(0, wpt)
    def _window(i):
        w = wid + i * _NW

        @pl.when(w < nwin)
        def _():
            r0 = w * _WROWS
            pltpu.sync_copy(src_hbm.at[pl.ds(r0, _WROWS)], idx_s)
            pltpu.sync_copy(dst_hbm.at[pl.ds(r0, _WROWS)], idx_d)

            handles = []
            for j in range(_WROWS):
                sl = pl.ds(j * 128, 128)
                handles.append(pltpu.async_copy(
                    coords_hbm.at[idx_s.at[j]], rows_s.at[sl], sem))
                handles.append(pltpu.async_copy(
                    coords_hbm.at[idx_d.at[j]], rows_d.at[sl], sem))
            for h in handles:
                h.wait()

            @pl.loop(0, _W, step=16)
            def _edges(e0):
                ri = iota16 + e0
                xs = plsc.load_gather(rows_s, [ri, col0])
                ys = plsc.load_gather(rows_s, [ri, col1])
                zs = plsc.load_gather(rows_s, [ri, col2])
                xd = plsc.load_gather(rows_d, [ri, col0])
                yd = plsc.load_gather(rows_d, [ri, col1])
                zd = plsc.load_gather(rows_d, [ri, col2])
                dx = xd - xs
                dy = yd - ys
                dz = zd - zs
                ss = jnp.maximum(dx * dx + dy * dy + dz * dz, 1e-24)
                yi = _MAGIC - lax.shift_right_logical(
                    plsc.bitcast(ss, jnp.int32), 1)
                y = plsc.bitcast(yi, jnp.float32)
                h = 0.5 * ss
                y = y * (1.5 - h * y * y)
                y = y * (1.5 - h * y * y)
                y = y * (1.5 - h * y * y)
                rr = lax.shift_right_logical(ri, 7)
                rc = ri & lmask
                plsc.store_scatter(dloc, [rr, rc], ss * y)
                plsc.store_scatter(vloc, [col0, rr, rc], dx * y)
                plsc.store_scatter(vloc, [col1, rr, rc], dy * y)
                plsc.store_scatter(vloc, [col2, rr, rc], dz * y)

            pltpu.sync_copy(dloc, d_hbm.at[pl.ds(w * _WROWS, _WROWS)])
            for c in range(3):
                pltpu.sync_copy(
                    vloc.at[c], vec_hbm.at[c, pl.ds(w * _WROWS, _WROWS)])


@functools.partial(jax.jit, static_argnames=())
def _sc_gather_norm(coords16, src2d, dst2d):
    E = src2d.shape[0] * 128
    mesh = plsc.VectorSubcoreMesh(core_axis_name="c", subcore_axis_name="s")
    cp = pltpu.CompilerParams()
    if "needs_layout_passes" in pltpu.CompilerParams.__dataclass_fields__:
        cp = dataclasses.replace(cp, needs_layout_passes=False)
    if "use_tc_tiling_on_sc" in pltpu.CompilerParams.__dataclass_fields__:
        cp = dataclasses.replace(cp, use_tc_tiling_on_sc=False)
    kern = pl.kernel(
        _sc_body,
        out_type=[
            jax.ShapeDtypeStruct((E // 128, 128), jnp.float32),
            jax.ShapeDtypeStruct((3, E // 128, 128), jnp.float32),
        ],
        mesh=mesh,
        scratch_types=[
            pltpu.VMEM((_WROWS, 128), jnp.int32),
            pltpu.VMEM((_WROWS, 128), jnp.int32),
            pltpu.VMEM((_W, 16), jnp.float32),
            pltpu.VMEM((_W, 16), jnp.float32),
            pltpu.VMEM((_WROWS, 128), jnp.float32),
            pltpu.VMEM((3, _WROWS, 128), jnp.float32),
            pltpu.SemaphoreType.DMA,
        ],
        compiler_params=cp,
    )
    return kern(coords16, src2d, dst2d)


def _rbf_body(d_ref, vec_ref, out_ref, vout_ref):
    dp = d_ref[...]
    bn = dp.shape[0]
    khi = pl.program_id(1)
    dd = jnp.broadcast_to(dp[:, None, :], (bn, 8, 128))
    klo = lax.broadcasted_iota(jnp.int32, (bn, 8, 128), 1)
    c = _C_STEP * (8 * khi + klo).astype(jnp.float32)
    t = dd - c
    out_ref[0] = jnp.exp(t * t * (-_INV_2W2))

    @pl.when(khi == 0)
    def _():
        vout_ref[...] = vec_ref[...]


def _rbf(d2d, vec3):
    R = d2d.shape[0]
    bn = 200
    return pl.pallas_call(
        _rbf_body,
        grid=(R // bn, 2),
        in_specs=[pl.BlockSpec((bn, 128), lambda i, k: (i, 0)),
                  pl.BlockSpec((3, bn, 128), lambda i, k: (0, i, 0))],
        out_specs=[pl.BlockSpec((1, bn, 8, 128), lambda i, k: (k, i, 0, 0)),
                   pl.BlockSpec((3, bn, 128), lambda i, k: (0, i, 0))],
        out_shape=[jax.ShapeDtypeStruct((2, R, 8, 128), jnp.float32),
                   jax.ShapeDtypeStruct((3, R, 128), jnp.float32)],
    )(d2d, vec3)


def kernel(coords, edge_index):
    N = coords.shape[0]
    E = edge_index.shape[1]
    coords16 = jnp.pad(coords, ((0, 0), (0, 13)))
    src2d = edge_index[0].reshape(E // 128, 128)
    dst2d = edge_index[1].reshape(E // 128, 128)
    d2d, vec3 = _sc_gather_norm(coords16, src2d, dst2d)
    scal4, vec3c = _rbf(d2d, vec3)
    scal = (scal4.transpose(0, 2, 1, 3)
            .reshape(_NUM_BASES, E).T)
    vec = vec3c.reshape(3, E).T.reshape(E, 3, 1)
    return scal, vec


# speedup vs baseline: 23.5904x; 1.1250x over previous
import dataclasses
import functools

import jax
import jax.numpy as jnp
import numpy as np
from jax import lax
from jax.experimental import pallas as pl
from jax.experimental.pallas import tpu as pltpu
from jax.experimental.pallas import tpu_sc as plsc

_NUM_BASES = 16
_D_MIN = 0.0
_D_MAX = 4.5
_WIDTH = (_D_MAX - _D_MIN) / _NUM_BASES
_INV_2W2 = 1.0 / (2.0 * _WIDTH * _WIDTH)
_C_STEP = (_D_MAX - _D_MIN) / (_NUM_BASES - 1)

_NC = 2
_NS = 16
_NW = _NC * _NS

_W = 1280
_WROWS = _W // 128

_MAGIC = np.int32(0x5F3759DF)


def _sc_body(coords_hbm, src_hbm, dst_hbm, d_hbm, vec_hbm,
             idx_s, idx_d, rows_s, rows_d, dloc, vloc, sem):
    E = src_hbm.shape[0] * 128
    nwin = E // _W
    wpt = (nwin + _NW - 1) // _NW
    cid = lax.axis_index("c")
    sid = lax.axis_index("s")
    wid = sid * _NC + cid

    iota16 = lax.iota(jnp.int32, 16)
    col0 = jnp.zeros((16,), jnp.int32)
    col1 = jnp.full((16,), 1, jnp.int32)
    col2 = jnp.full((16,), 2, jnp.int32)
    lmask = jnp.full((16,), 127, jnp.int32)

    @pl.loop(0, wpt)
    def _window(i):
        w = wid + i * _NW

        @pl.when(w < nwin)
        def _():
            r0 = w * _WROWS
            pltpu.sync_copy(src_hbm.at[pl.ds(r0, _WROWS)], idx_s)
            pltpu.sync_copy(dst_hbm.at[pl.ds(r0, _WROWS)], idx_d)

            handles = []
            for j in range(_WROWS):
                sl = pl.ds(j * 128, 128)
                handles.append(pltpu.async_copy(
                    coords_hbm.at[idx_s.at[j]], rows_s.at[sl], sem))
                handles.append(pltpu.async_copy(
                    coords_hbm.at[idx_d.at[j]], rows_d.at[sl], sem))
            for h in handles:
                h.wait()

            @pl.loop(0, _W, step=16)
            def _edges(e0):
                ri = iota16 + e0
                xs = plsc.load_gather(rows_s, [ri, col0])
                ys = plsc.load_gather(rows_s, [ri, col1])
                zs = plsc.load_gather(rows_s, [ri, col2])
                xd = plsc.load_gather(rows_d, [ri, col0])
                yd = plsc.load_gather(rows_d, [ri, col1])
                zd = plsc.load_gather(rows_d, [ri, col2])
                dx = xd - xs
                dy = yd - ys
                dz = zd - zs
                ss = jnp.maximum(dx * dx + dy * dy + dz * dz, 1e-24)
                yi = _MAGIC - lax.shift_right_logical(
                    plsc.bitcast(ss, jnp.int32), 1)
                y = plsc.bitcast(yi, jnp.float32)
                h = 0.5 * ss
                y = y * (1.5 - h * y * y)
                y = y * (1.5 - h * y * y)
                y = y * (1.5 - h * y * y)
                rr = lax.shift_right_logical(ri, 7)
                rc = ri & lmask
                plsc.store_scatter(dloc, [rr, rc], ss * y)
                plsc.store_scatter(vloc, [col0, rr, rc], dx * y)
                plsc.store_scatter(vloc, [col1, rr, rc], dy * y)
                plsc.store_scatter(vloc, [col2, rr, rc], dz * y)

            pltpu.sync_copy(dloc, d_hbm.at[pl.ds(w * _WROWS, _WROWS)])
            for c in range(3):
                pltpu.sync_copy(
                    vloc.at[c], vec_hbm.at[c, pl.ds(w * _WROWS, _WROWS)])


@functools.partial(jax.jit, static_argnames=())
def _sc_gather_norm(coords16, src2d, dst2d):
    E = src2d.shape[0] * 128
    mesh = plsc.VectorSubcoreMesh(core_axis_name="c", subcore_axis_name="s")
    cp = pltpu.CompilerParams()
    if "needs_layout_passes" in pltpu.CompilerParams.__dataclass_fields__:
        cp = dataclasses.replace(cp, needs_layout_passes=False)
    if "use_tc_tiling_on_sc" in pltpu.CompilerParams.__dataclass_fields__:
        cp = dataclasses.replace(cp, use_tc_tiling_on_sc=False)
    kern = pl.kernel(
        _sc_body,
        out_type=[
            jax.ShapeDtypeStruct((E // 128, 128), jnp.float32),
            jax.ShapeDtypeStruct((3, E // 128, 128), jnp.float32),
        ],
        mesh=mesh,
        scratch_types=[
            pltpu.VMEM((_WROWS, 128), jnp.int32),
            pltpu.VMEM((_WROWS, 128), jnp.int32),
            pltpu.VMEM((_W, 16), jnp.float32),
            pltpu.VMEM((_W, 16), jnp.float32),
            pltpu.VMEM((_WROWS, 128), jnp.float32),
            pltpu.VMEM((3, _WROWS, 128), jnp.float32),
            pltpu.SemaphoreType.DMA,
        ],
        compiler_params=cp,
    )
    return kern(coords16, src2d, dst2d)


def _rbf_body(d_ref, vec_ref, out_ref, vout_ref):
    dp = d_ref[...]
    bn = dp.shape[0]
    khi = pl.program_id(1)
    dd = jnp.broadcast_to(dp[:, None, :], (bn, 8, 128))
    klo = lax.broadcasted_iota(jnp.int32, (bn, 8, 128), 1)
    c = _C_STEP * (8 * khi + klo).astype(jnp.float32)
    t = dd - c
    out_ref[0] = jnp.exp(t * t * (-_INV_2W2))

    @pl.when(khi == 0)
    def _():
        vout_ref[...] = vec_ref[...]


def _rbf(d2d, vec3):
    R = d2d.shape[0]
    bn = 200
    return pl.pallas_call(
        _rbf_body,
        grid=(R // bn, 2),
        in_specs=[pl.BlockSpec((bn, 128), lambda i, k: (i, 0)),
                  pl.BlockSpec((3, bn, 128), lambda i, k: (0, i, 0))],
        out_specs=[pl.BlockSpec((1, bn, 8, 128), lambda i, k: (k, i, 0, 0)),
                   pl.BlockSpec((3, bn, 128), lambda i, k: (0, i, 0))],
        out_shape=[jax.ShapeDtypeStruct((2, R, 8, 128), jnp.float32),
                   jax.ShapeDtypeStruct((3, R, 128), jnp.float32)],
    )(d2d, vec3)


def kernel(coords, edge_index):
    N = coords.shape[0]
    E = edge_index.shape[1]
    coords16 = jnp.pad(coords, ((0, 0), (0, 13)))
    src2d = edge_index[0].reshape(E // 128, 128)
    dst2d = edge_index[1].reshape(E // 128, 128)
    d2d, vec3 = _sc_gather_norm(coords16, src2d, dst2d)
    scal4, vec3c = _rbf(d2d, vec3)
    scal = (scal4.transpose(0, 2, 1, 3)
            .reshape(_NUM_BASES, E).T)
    vec = vec3c.reshape(3, E).T.reshape(E, 3, 1)
    return scal, vec


# speedup vs baseline: 25.0584x; 1.0622x over previous
import dataclasses
import functools

import jax
import jax.numpy as jnp
import numpy as np
from jax import lax
from jax.experimental import pallas as pl
from jax.experimental.pallas import tpu as pltpu
from jax.experimental.pallas import tpu_sc as plsc

_NUM_BASES = 16
_D_MIN = 0.0
_D_MAX = 4.5
_WIDTH = (_D_MAX - _D_MIN) / _NUM_BASES
_INV_2W2 = 1.0 / (2.0 * _WIDTH * _WIDTH)
_C_STEP = (_D_MAX - _D_MIN) / (_NUM_BASES - 1)

_NC = 2
_NS = 16
_NW = _NC * _NS

_W = 2560
_WROWS = _W // 128

_MAGIC = np.int32(0x5F3759DF)


def _sc_body(coords_hbm, src_hbm, dst_hbm, d_hbm, vec_hbm,
             idx_s, idx_d, rows_s, rows_d, dloc, vloc, sem):
    E = src_hbm.shape[0] * 128
    nwin = E // _W
    wpt = (nwin + _NW - 1) // _NW
    cid = lax.axis_index("c")
    sid = lax.axis_index("s")
    wid = sid * _NC + cid

    iota16 = lax.iota(jnp.int32, 16)
    col0 = jnp.zeros((16,), jnp.int32)
    col1 = jnp.full((16,), 1, jnp.int32)
    col2 = jnp.full((16,), 2, jnp.int32)
    lmask = jnp.full((16,), 127, jnp.int32)

    @pl.loop(0, wpt)
    def _window(i):
        w = wid + i * _NW

        @pl.when(w < nwin)
        def _():
            r0 = w * _WROWS
            pltpu.sync_copy(src_hbm.at[pl.ds(r0, _WROWS)], idx_s)
            pltpu.sync_copy(dst_hbm.at[pl.ds(r0, _WROWS)], idx_d)

            handles = []
            for j in range(_WROWS):
                sl = pl.ds(j * 128, 128)
                handles.append(pltpu.async_copy(
                    coords_hbm.at[idx_s.at[j]], rows_s.at[sl], sem))
                handles.append(pltpu.async_copy(
                    coords_hbm.at[idx_d.at[j]], rows_d.at[sl], sem))
            for h in handles:
                h.wait()

            @pl.loop(0, _W, step=16)
            def _edges(e0):
                ri = iota16 + e0
                xs = plsc.load_gather(rows_s, [ri, col0])
                ys = plsc.load_gather(rows_s, [ri, col1])
                zs = plsc.load_gather(rows_s, [ri, col2])
                xd = plsc.load_gather(rows_d, [ri, col0])
                yd = plsc.load_gather(rows_d, [ri, col1])
                zd = plsc.load_gather(rows_d, [ri, col2])
                dx = xd - xs
                dy = yd - ys
                dz = zd - zs
                ss = jnp.maximum(dx * dx + dy * dy + dz * dz, 1e-24)
                yi = _MAGIC - lax.shift_right_logical(
                    plsc.bitcast(ss, jnp.int32), 1)
                y = plsc.bitcast(yi, jnp.float32)
                h = 0.5 * ss
                y = y * (1.5 - h * y * y)
                y = y * (1.5 - h * y * y)
                y = y * (1.5 - h * y * y)
                rr = lax.shift_right_logical(ri, 7)
                rc = ri & lmask
                plsc.store_scatter(dloc, [rr, rc], ss * y)
                plsc.store_scatter(vloc, [col0, rr, rc], dx * y)
                plsc.store_scatter(vloc, [col1, rr, rc], dy * y)
                plsc.store_scatter(vloc, [col2, rr, rc], dz * y)

            pltpu.sync_copy(dloc, d_hbm.at[pl.ds(w * _WROWS, _WROWS)])
            for c in range(3):
                pltpu.sync_copy(
                    vloc.at[c], vec_hbm.at[c, pl.ds(w * _WROWS, _WROWS)])


@functools.partial(jax.jit, static_argnames=())
def _sc_gather_norm(coords16, src2d, dst2d):
    E = src2d.shape[0] * 128
    mesh = plsc.VectorSubcoreMesh(core_axis_name="c", subcore_axis_name="s")
    cp = pltpu.CompilerParams()
    if "needs_layout_passes" in pltpu.CompilerParams.__dataclass_fields__:
        cp = dataclasses.replace(cp, needs_layout_passes=False)
    if "use_tc_tiling_on_sc" in pltpu.CompilerParams.__dataclass_fields__:
        cp = dataclasses.replace(cp, use_tc_tiling_on_sc=False)
    kern = pl.kernel(
        _sc_body,
        out_type=[
            jax.ShapeDtypeStruct((E // 128, 128), jnp.float32),
            jax.ShapeDtypeStruct((3, E // 128, 128), jnp.float32),
        ],
        mesh=mesh,
        scratch_types=[
            pltpu.VMEM((_WROWS, 128), jnp.int32),
            pltpu.VMEM((_WROWS, 128), jnp.int32),
            pltpu.VMEM((_W, 16), jnp.float32),
            pltpu.VMEM((_W, 16), jnp.float32),
            pltpu.VMEM((_WROWS, 128), jnp.float32),
            pltpu.VMEM((3, _WROWS, 128), jnp.float32),
            pltpu.SemaphoreType.DMA,
        ],
        compiler_params=cp,
    )
    return kern(coords16, src2d, dst2d)


def _rbf_body(d_ref, vec_ref, out_ref, vout_ref):
    dp = d_ref[...]
    bn = dp.shape[0]
    khi = pl.program_id(1)
    dd = jnp.broadcast_to(dp[:, None, :], (bn, 8, 128))
    klo = lax.broadcasted_iota(jnp.int32, (bn, 8, 128), 1)
    c = _C_STEP * (8 * khi + klo).astype(jnp.float32)
    t = dd - c
    out_ref[0] = jnp.exp(t * t * (-_INV_2W2))

    @pl.when(khi == 0)
    def _():
        vout_ref[...] = vec_ref[...]


def _rbf(d2d, vec3):
    R = d2d.shape[0]
    bn = 200
    return pl.pallas_call(
        _rbf_body,
        grid=(R // bn, 2),
        in_specs=[pl.BlockSpec((bn, 128), lambda i, k: (i, 0)),
                  pl.BlockSpec((3, bn, 128), lambda i, k: (0, i, 0))],
        out_specs=[pl.BlockSpec((1, bn, 8, 128), lambda i, k: (k, i, 0, 0)),
                   pl.BlockSpec((3, bn, 128), lambda i, k: (0, i, 0))],
        out_shape=[jax.ShapeDtypeStruct((2, R, 8, 128), jnp.float32),
                   jax.ShapeDtypeStruct((3, R, 128), jnp.float32)],
    )(d2d, vec3)


def kernel(coords, edge_index):
    N = coords.shape[0]
    E = edge_index.shape[1]
    coords16 = jnp.pad(coords, ((0, 0), (0, 13)))
    src2d = edge_index[0].reshape(E // 128, 128)
    dst2d = edge_index[1].reshape(E // 128, 128)
    d2d, vec3 = _sc_gather_norm(coords16, src2d, dst2d)
    scal4, vec3c = _rbf(d2d, vec3)
    scal = (scal4.transpose(0, 2, 1, 3)
            .reshape(_NUM_BASES, E).T)
    vec = vec3c.reshape(3, E).T.reshape(E, 3, 1)
    return scal, vec
